# baseline (speedup 1.0000x reference)
import functools

import jax
import jax.numpy as jnp
from jax import lax
from jax.experimental import pallas as pl
from jax.experimental.pallas import tpu as pltpu
from jax.experimental.pallas import tpu_sc as plsc

LAT = 128
N_PAD = 10240
BN = 1024
BE = 2000
E_TOT = 320000


def _ln(u):
    mu = jnp.mean(u, axis=-1, keepdims=True)
    d = u - mu
    var = jnp.mean(d * d, axis=-1, keepdims=True)
    return d * jax.lax.rsqrt(var + 1e-5)


def _mm(a, b):
    return jax.lax.dot_general(a, b, (((1,), (0,)), ((), ())),
                               preferred_element_type=jnp.float32)


def _enc_node_body(x, w0, b0, w1, b1, w2, b2, w0r, w0s, xl_o, ar_o, as_o):
    h = jnp.maximum(_mm(x[...], w0[...]) + b0[...], 0.0)
    h = jnp.maximum(_mm(h, w1[...]) + b1[...], 0.0)
    xl = _ln(_mm(h, w2[...]) + b2[...])
    xl_o[...] = xl
    ar_o[...] = _mm(xl, w0r[...]).astype(jnp.bfloat16)
    as_o[...] = _mm(xl, w0s[...]).astype(jnp.bfloat16)


def _enc_edge_body(ea, w0, b0, w1, b1, w2, b2, el_o):
    h = jnp.maximum(_mm(ea[...], w0[...]) + b0[...], 0.0)
    h = jnp.maximum(_mm(h, w1[...]) + b1[...], 0.0)
    el_o[...] = _ln(_mm(h, w2[...]) + b2[...])


def _edge_body(has_prev, *refs):
    if has_prev:
        (el, ue, gr, gs, w0, b0, w1, b1, w2, b2, out) = refs
        pe = el[...] + ue[...]
    else:
        (el, gr, gs, w0, b0, w1, b1, w2, b2, out) = refs
        pe = el[...]
    g = gr[...].astype(jnp.float32) + gs[...].astype(jnp.float32)
    h = jnp.maximum(_mm(pe, w0[...]) + g + b0[...], 0.0)
    h = jnp.maximum(_mm(h, w1[...]) + b1[...], 0.0)
    out[...] = _ln(_mm(h, w2[...]) + b2[...])


def _node_body(px, xl, agg0, agg1, w0a, w0b, b0, w1, b1, w2, b2, w0r, w0s,
               px_o, ar_o, as_o):
    agg = agg0[...] + agg1[...]
    h = jnp.maximum(_mm(px[...], w0a[...]) + _mm(agg, w0b[...]) + b0[...],
                    0.0)
    h = jnp.maximum(_mm(h, w1[...]) + b1[...], 0.0)
    u = _ln(_mm(h, w2[...]) + b2[...])
    px_new = xl[...] + u
    px_o[...] = px_new
    ar_o[...] = _mm(px_new, w0r[...]).astype(jnp.bfloat16)
    as_o[...] = _mm(px_new, w0s[...]).astype(jnp.bfloat16)


def _dec_body(px, w0, b0, w1, b1, w2, b2, out):
    h = jnp.maximum(_mm(px[...], w0[...]) + b0[...], 0.0)
    h = jnp.maximum(_mm(h, w1[...]) + b1[...], 0.0)
    out[...] = _mm(h, w2[...]) + b2[...]


def _row_spec(bm, d):
    return pl.BlockSpec((bm, d), lambda i: (i, 0))


def _full_spec(shape):
    return pl.BlockSpec(shape, lambda i: tuple(0 for _ in shape))


def _w_specs(shapes):
    return [_full_spec(s) for s in shapes]


def _call_rows(body, n_rows, bm, row_ins, full_ins, out_dtypes, out_d=LAT):
    grid = n_rows // bm
    n_out = len(out_dtypes)
    in_specs = ([_row_spec(bm, a.shape[-1]) for a in row_ins]
                + _w_specs([a.shape for a in full_ins]))
    out_specs = [_row_spec(bm, out_d) for _ in range(n_out)]
    out_shape = [jax.ShapeDtypeStruct((n_rows, out_d), dt)
                 for dt in out_dtypes]
    if n_out == 1:
        out_specs, out_shape = out_specs[0], out_shape[0]
    return pl.pallas_call(
        body,
        grid=(grid,),
        in_specs=in_specs,
        out_specs=out_specs,
        out_shape=out_shape,
    )(*row_ins, *full_ins)


def _prep_mlp(p):
    return (p["w0"], p["b0"].reshape(1, -1), p["w1"], p["b1"].reshape(1, -1),
            p["w2"], p["b2"].reshape(1, -1))


_NC, _NS = 2, 16
_NW = _NC * _NS
_C = 128
_NCHUNK = E_TOT // _C
_JMAX = -(-_NCHUNK // _NW)
_NCHUNK_PAD = _JMAX * _NW
_K = 4
_JMAXP = _K * (-(-_JMAX // _K))
_RPT = N_PAD // _NS
_LATP = LAT // 2
_JMAX1 = -(-_NCHUNK // _NS)
_NCHUNK_PAD1 = _JMAX1 * _NS
_KS = 2
_JMAXP1 = _KS * (-(-_JMAX1 // _KS))


def _sc_mesh(num_cores=2):
    return plsc.VectorSubcoreMesh(core_axis_name="c", subcore_axis_name="s",
                                  num_cores=num_cores)


def _gather(ar, as_, rcvt, sndt):
    @functools.partial(
        pl.kernel,
        out_type=[jax.ShapeDtypeStruct((E_TOT, _LATP), jnp.float32),
                  jax.ShapeDtypeStruct((E_TOT, _LATP), jnp.float32)],
        mesh=_sc_mesh(),
        scratch_types=[
            pltpu.VMEM((_JMAXP, _C), jnp.int32),
            pltpu.VMEM((_JMAXP, _C), jnp.int32),
            pltpu.VMEM((_K, _C, _LATP), jnp.float32),
            pltpu.VMEM((_K, _C, _LATP), jnp.float32),
        ] + [pltpu.SemaphoreType.DMA] * (2 * _K),
        compiler_params=pltpu.CompilerParams(use_tc_tiling_on_sc=False),
    )
    def k(ar_h, as_h, rcv_h, snd_h, gr_h, gs_h, ir, is_, rr, rs, *sems):
        w = lax.axis_index("s") * _NC + lax.axis_index("c")
        pltpu.sync_copy(rcv_h.at[w], ir.at[pl.ds(0, _JMAX)])
        pltpu.sync_copy(snd_h.at[w], is_.at[pl.ds(0, _JMAX)])
        zi = jnp.zeros((16,), jnp.int32)
        for jj in range(_C // 16):
            for j0 in range(_JMAX, _JMAXP):
                ir[j0, pl.ds(jj * 16, 16)] = zi
                is_[j0, pl.ds(jj * 16, 16)] = zi

        def fire(j, b):
            pltpu.async_copy(ar_h.at[ir.at[j]], rr.at[b], sems[b])
            pltpu.async_copy(as_h.at[is_.at[j]], rs.at[b], sems[_K + b])

        def drain(b):
            pltpu.make_async_copy(
                ar_h.at[pl.ds(0, _C)], rr.at[b], sems[b]).wait()
            pltpu.make_async_copy(
                as_h.at[pl.ds(0, _C)], rs.at[b], sems[_K + b]).wait()

        for b in range(_K):
            fire(b, b)

        def body(g, carry):
            for b in range(_K):
                j = g * _K + b
                drain(b)
                c = w + _NW * j

                @pl.when(c < _NCHUNK)
                def _():
                    pltpu.sync_copy(rr.at[b], gr_h.at[pl.ds(c * _C, _C)])
                    pltpu.sync_copy(rs.at[b], gs_h.at[pl.ds(c * _C, _C)])

                jn = j + _K

                @pl.when(jn < _JMAXP)
                def _():
                    fire(jn, b)
            return carry

        lax.fori_loop(0, _JMAXP // _K, body, 0)

    return k(ar, as_, rcvt, sndt)


def _scatter(upd_e, rcv2):
    @functools.partial(
        pl.kernel,
        out_type=[jax.ShapeDtypeStruct((N_PAD, LAT), jnp.float32),
                  jax.ShapeDtypeStruct((N_PAD, LAT), jnp.float32)],
        mesh=_sc_mesh(),
        scratch_types=[
            pltpu.VMEM((_C,), jnp.int32),
            pltpu.VMEM((_C, LAT), jnp.float32),
            pltpu.VMEM_SHARED((N_PAD, LAT), jnp.float32),
        ],
    )
    def k(ue_h, rcv_h, out0_h, out1_h, idx_v, rows_v, agg_sh):
        cidx = lax.axis_index("c")
        sidx = lax.axis_index("s")
        w = sidx * _NC + cidx
        zero16 = jnp.zeros((16,), jnp.float32)

        def zbody(i, carry):
            for jj in range(LAT // 16):
                rows_v[i, pl.ds(jj * 16, 16)] = zero16
            return carry

        lax.fori_loop(0, _C, zbody, 0)
        for t in range(_RPT // _C):
            pltpu.sync_copy(rows_v, agg_sh.at[pl.ds(sidx * _RPT + t * _C, _C)])
        plsc.subcore_barrier()

        def body(j, carry):
            c = w + _NW * j

            @pl.when(c < _NCHUNK)
            def _():
                pltpu.sync_copy(rcv_h.at[c], idx_v)
                pltpu.sync_copy(ue_h.at[pl.ds(c * _C, _C)], rows_v)
                pltpu.sync_copy(rows_v, agg_sh.at[idx_v], add=True)
            return carry

        lax.fori_loop(0, _JMAX, body, 0)
        plsc.subcore_barrier()

        for t in range(_RPT // _C):
            lo = sidx * _RPT + t * _C
            pltpu.sync_copy(agg_sh.at[pl.ds(lo, _C)], rows_v)

            @pl.when(cidx == 0)
            def _():
                pltpu.sync_copy(rows_v, out0_h.at[pl.ds(lo, _C)])

            @pl.when(cidx == 1)
            def _():
                pltpu.sync_copy(rows_v, out1_h.at[pl.ds(lo, _C)])

    return k(upd_e, rcv2)


def _tile_major(idx, nw, jmax):
    a = jnp.zeros((jmax * nw * _C,), jnp.int32)
    a = a.at[:E_TOT].set(idx.astype(jnp.int32))
    return a.reshape(jmax, nw, _C).transpose(1, 0, 2)


def _pack_bf16(a):
    return lax.bitcast_convert_type(
        a.reshape(a.shape[0], _LATP, 2), jnp.float32)


def _unpack_bf16(a):
    return lax.bitcast_convert_type(a, jnp.bfloat16).reshape(a.shape[0], LAT)


def kernel(x, edge_attr, receivers, senders, params):
    N = x.shape[0]
    x_p = jnp.pad(x, ((0, N_PAD - N), (0, 0)))
    rcvt_g = _tile_major(receivers, _NW, _JMAX)
    sndt_g = _tile_major(senders, _NW, _JMAX)
    rcv2 = receivers.astype(jnp.int32).reshape(_NCHUNK, _C)

    pe0 = params["procs"][0]["edge"]
    w0r0 = pe0["w0"][LAT:2 * LAT]
    w0s0 = pe0["w0"][2 * LAT:]

    en = _prep_mlp(params["enc_node"])
    x_l, ar, as_ = _call_rows(
        _enc_node_body, N_PAD, BN, [x_p], [*en, w0r0, w0s0],
        [jnp.float32, jnp.bfloat16, jnp.bfloat16])

    ee = _prep_mlp(params["enc_edge"])
    e_l = _call_rows(_enc_edge_body, E_TOT, BE, [edge_attr], [*ee],
                     [jnp.float32])

    pre_x = x_l
    upd_e = None
    for i, p in enumerate(params["procs"]):
        pedge = _prep_mlp(p["edge"])
        w0e = pedge[0][:LAT]
        grp, gsp = _gather(_pack_bf16(ar), _pack_bf16(as_), rcvt_g, sndt_g)
        gr, gs = _unpack_bf16(grp), _unpack_bf16(gsp)
        if upd_e is None:
            upd_e = _call_rows(
                functools.partial(_edge_body, False), E_TOT, BE,
                [e_l, gr, gs], [w0e, *pedge[1:]], [jnp.float32])
        else:
            upd_e = _call_rows(
                functools.partial(_edge_body, True), E_TOT, BE,
                [e_l, upd_e, gr, gs], [w0e, *pedge[1:]], [jnp.float32])
        agg0, agg1 = _scatter(upd_e, rcv2)

        pnode = _prep_mlp(p["node"])
        w0a = pnode[0][:LAT]
        w0b = pnode[0][LAT:]
        if i + 1 < len(params["procs"]):
            pe_next = params["procs"][i + 1]["edge"]["w0"]
            w0rn = pe_next[LAT:2 * LAT]
            w0sn = pe_next[2 * LAT:]
        else:
            w0rn = w0r0
            w0sn = w0s0
        pre_x, ar, as_ = _call_rows(
            _node_body, N_PAD, BN,
            [pre_x, x_l, agg0, agg1], [w0a, w0b, *pnode[1:], w0rn, w0sn],
            [jnp.float32, jnp.bfloat16, jnp.bfloat16])

    dec = _prep_mlp(params["dec"])
    w2d = jnp.pad(dec[4], ((0, 0), (0, LAT - dec[4].shape[1])))
    b2d = jnp.pad(dec[5], ((0, 0), (0, LAT - dec[5].shape[1])))
    out = _call_rows(_dec_body, N_PAD, BN,
                     [pre_x], [dec[0], dec[1], dec[2], dec[3], w2d, b2d],
                     [jnp.float32])
    return out[:N, :3]


# speedup vs baseline: 3.3194x; 3.3194x over previous
import functools

import jax
import jax.numpy as jnp
from jax import lax
from jax.experimental import pallas as pl
from jax.experimental.pallas import tpu as pltpu
from jax.experimental.pallas import tpu_sc as plsc

LAT = 128
N_PAD = 10240
BN = 1024
BE = 2000
E_TOT = 320000
E_HALF = E_TOT // 2


def _ln(u):
    mu = jnp.mean(u, axis=-1, keepdims=True)
    d = u - mu
    var = jnp.mean(d * d, axis=-1, keepdims=True)
    return d * jax.lax.rsqrt(var + 1e-5)


def _mm(a, b):
    return jax.lax.dot_general(a, b, (((1,), (0,)), ((), ())),
                               preferred_element_type=jnp.float32)


def _enc_node_body(x, w0, b0, w1, b1, w2, b2, w0r, w0s, xl_o, ar_o, as_o):
    h = jnp.maximum(_mm(x[...], w0[...]) + b0[...], 0.0)
    h = jnp.maximum(_mm(h, w1[...]) + b1[...], 0.0)
    xl = _ln(_mm(h, w2[...]) + b2[...])
    xl_o[...] = xl
    ar_o[...] = _mm(xl, w0r[...])
    as_o[...] = _mm(xl, w0s[...])


def _enc_edge_body(ea, w0, b0, w1, b1, w2, b2, el_o):
    h = jnp.maximum(_mm(ea[...], w0[...]) + b0[...], 0.0)
    h = jnp.maximum(_mm(h, w1[...]) + b1[...], 0.0)
    el_o[...] = _ln(_mm(h, w2[...]) + b2[...])


def _edge_body(has_prev, *refs):
    if has_prev:
        (el, ue, gr, gs, w0, b0, w1, b1, w2, b2, out) = refs
        pe = el[...] + ue[...]
    else:
        (el, gr, gs, w0, b0, w1, b1, w2, b2, out) = refs
        pe = el[...]
    g = gr[...] + gs[...]
    h = jnp.maximum(_mm(pe, w0[...]) + g + b0[...], 0.0)
    h = jnp.maximum(_mm(h, w1[...]) + b1[...], 0.0)
    out[...] = _ln(_mm(h, w2[...]) + b2[...])


def _node_body(px, xl, a0, a1, a2, a3, w0a, w0b, b0, w1, b1, w2, b2,
               w0r, w0s, px_o, ar_o, as_o):
    agg = (a0[...] + a1[...]) + (a2[...] + a3[...])
    h = jnp.maximum(_mm(px[...], w0a[...]) + _mm(agg, w0b[...]) + b0[...],
                    0.0)
    h = jnp.maximum(_mm(h, w1[...]) + b1[...], 0.0)
    u = _ln(_mm(h, w2[...]) + b2[...])
    px_new = xl[...] + u
    px_o[...] = px_new
    ar_o[...] = _mm(px_new, w0r[...])
    as_o[...] = _mm(px_new, w0s[...])


def _dec_body(px, w0, b0, w1, b1, w2, b2, out):
    h = jnp.maximum(_mm(px[...], w0[...]) + b0[...], 0.0)
    h = jnp.maximum(_mm(h, w1[...]) + b1[...], 0.0)
    out[...] = _mm(h, w2[...]) + b2[...]


def _row_spec(bm, d):
    return pl.BlockSpec((bm, d), lambda i: (i, 0))


def _full_spec(shape):
    return pl.BlockSpec(shape, lambda i: tuple(0 for _ in shape))


def _call_rows(body, n_rows, bm, row_ins, full_ins, out_dtypes):
    grid = n_rows // bm
    n_out = len(out_dtypes)
    in_specs = ([_row_spec(bm, a.shape[-1]) for a in row_ins]
                + [_full_spec(a.shape) for a in full_ins])
    out_specs = [_row_spec(bm, LAT) for _ in range(n_out)]
    out_shape = [jax.ShapeDtypeStruct((n_rows, LAT), dt) for dt in out_dtypes]
    if n_out == 1:
        out_specs, out_shape = out_specs[0], out_shape[0]
    return pl.pallas_call(
        body,
        grid=(grid,),
        in_specs=in_specs,
        out_specs=out_specs,
        out_shape=out_shape,
    )(*row_ins, *full_ins)


def _prep_mlp(p):
    return (p["w0"], p["b0"].reshape(1, -1), p["w1"], p["b1"].reshape(1, -1),
            p["w2"], p["b2"].reshape(1, -1))


_NC, _NS = 2, 16
_NW = _NC * _NS
_C = 128
_NCH = E_HALF // _C
_JH = -(-_NCH // _NW)
_RPT = N_PAD // _NS


def _sc_mesh():
    return plsc.VectorSubcoreMesh(core_axis_name="c", subcore_axis_name="s")


def _gather_half(ar, as_, rcv2, snd2):
    @functools.partial(
        pl.kernel,
        out_type=[jax.ShapeDtypeStruct((E_HALF, LAT), jnp.float32),
                  jax.ShapeDtypeStruct((E_HALF, LAT), jnp.float32)],
        mesh=_sc_mesh(),
        scratch_types=[
            pltpu.VMEM((_C,), jnp.int32),
            pltpu.VMEM((_C,), jnp.int32),
            pltpu.VMEM((_C, LAT), jnp.float32),
            pltpu.VMEM((_C, LAT), jnp.float32),
            pltpu.SemaphoreType.DMA,
            pltpu.SemaphoreType.DMA,
        ],
    )
    def k(ar_h, as_h, rcv_h, snd_h, gr_h, gs_h, ir, is_, rr, rs, sr, ss):
        w = lax.axis_index("s") * _NC + lax.axis_index("c")

        def body(j, carry):
            c = w + _NW * j

            @pl.when(c < _NCH)
            def _():
                pltpu.sync_copy(rcv_h.at[c], ir)
                pltpu.sync_copy(snd_h.at[c], is_)
                d1 = pltpu.async_copy(ar_h.at[ir], rr, sr)
                d2 = pltpu.async_copy(as_h.at[is_], rs, ss)
                d1.wait()
                d2.wait()
                pltpu.sync_copy(rr, gr_h.at[pl.ds(c * _C, _C)])
                pltpu.sync_copy(rs, gs_h.at[pl.ds(c * _C, _C)])
            return carry

        lax.fori_loop(0, _JH, body, 0)

    return k(ar, as_, rcv2, snd2)


def _scatter_half(upd_e, rcv2):
    @functools.partial(
        pl.kernel,
        out_type=[jax.ShapeDtypeStruct((N_PAD, LAT), jnp.float32),
                  jax.ShapeDtypeStruct((N_PAD, LAT), jnp.float32)],
        mesh=_sc_mesh(),
        scratch_types=[
            pltpu.VMEM((_C,), jnp.int32),
            pltpu.VMEM((_C, LAT), jnp.float32),
            pltpu.VMEM_SHARED((N_PAD, LAT), jnp.float32),
        ],
    )
    def k(ue_h, rcv_h, out0_h, out1_h, idx_v, rows_v, agg_sh):
        cidx = lax.axis_index("c")
        sidx = lax.axis_index("s")
        w = sidx * _NC + cidx
        zero16 = jnp.zeros((16,), jnp.float32)

        def zbody(i, carry):
            for jj in range(LAT // 16):
                rows_v[i, pl.ds(jj * 16, 16)] = zero16
            return carry

        lax.fori_loop(0, _C, zbody, 0)
        for t in range(_RPT // _C):
            pltpu.sync_copy(rows_v, agg_sh.at[pl.ds(sidx * _RPT + t * _C, _C)])
        plsc.subcore_barrier()

        def body(j, carry):
            c = w + _NW * j

            @pl.when(c < _NCH)
            def _():
                pltpu.sync_copy(rcv_h.at[c], idx_v)
                pltpu.sync_copy(ue_h.at[pl.ds(c * _C, _C)], rows_v)
                pltpu.sync_copy(rows_v, agg_sh.at[idx_v], add=True)
            return carry

        lax.fori_loop(0, _JH, body, 0)
        plsc.subcore_barrier()

        for t in range(_RPT // _C):
            lo = sidx * _RPT + t * _C
            pltpu.sync_copy(agg_sh.at[pl.ds(lo, _C)], rows_v)

            @pl.when(cidx == 0)
            def _():
                pltpu.sync_copy(rows_v, out0_h.at[pl.ds(lo, _C)])

            @pl.when(cidx == 1)
            def _():
                pltpu.sync_copy(rows_v, out1_h.at[pl.ds(lo, _C)])

    return k(upd_e, rcv2)


def kernel(x, edge_attr, receivers, senders, params):
    N = x.shape[0]
    x_p = jnp.pad(x, ((0, N_PAD - N), (0, 0)))
    rcv = receivers.astype(jnp.int32)
    snd = senders.astype(jnp.int32)
    rcv2 = [rcv[:E_HALF].reshape(_NCH, _C), rcv[E_HALF:].reshape(_NCH, _C)]
    snd2 = [snd[:E_HALF].reshape(_NCH, _C), snd[E_HALF:].reshape(_NCH, _C)]
    ea = [edge_attr[:E_HALF], edge_attr[E_HALF:]]

    pe0 = params["procs"][0]["edge"]
    w0r0 = pe0["w0"][LAT:2 * LAT]
    w0s0 = pe0["w0"][2 * LAT:]

    en = _prep_mlp(params["enc_node"])
    x_l, ar, as_ = _call_rows(
        _enc_node_body, N_PAD, BN, [x_p], [*en, w0r0, w0s0],
        [jnp.float32, jnp.float32, jnp.float32])

    ee = _prep_mlp(params["enc_edge"])
    e_l = [_call_rows(_enc_edge_body, E_HALF, BE, [ea[h]], [*ee],
                      [jnp.float32]) for h in range(2)]

    pre_x = x_l
    upd_e = [None, None]
    for i, p in enumerate(params["procs"]):
        pedge = _prep_mlp(p["edge"])
        w0e = pedge[0][:LAT]
        g = [_gather_half(ar, as_, rcv2[h], snd2[h]) for h in range(2)]
        aggs = []
        for h in range(2):
            gr, gs = g[h]
            if upd_e[h] is None:
                upd_e[h] = _call_rows(
                    functools.partial(_edge_body, False), E_HALF, BE,
                    [e_l[h], gr, gs], [w0e, *pedge[1:]], [jnp.float32])
            else:
                upd_e[h] = _call_rows(
                    functools.partial(_edge_body, True), E_HALF, BE,
                    [e_l[h], upd_e[h], gr, gs], [w0e, *pedge[1:]],
                    [jnp.float32])
            aggs.extend(_scatter_half(upd_e[h], rcv2[h]))

        pnode = _prep_mlp(p["node"])
        w0a = pnode[0][:LAT]
        w0b = pnode[0][LAT:]
        if i + 1 < len(params["procs"]):
            pe_next = params["procs"][i + 1]["edge"]["w0"]
            w0rn = pe_next[LAT:2 * LAT]
            w0sn = pe_next[2 * LAT:]
        else:
            w0rn = w0r0
            w0sn = w0s0
        pre_x, ar, as_ = _call_rows(
            _node_body, N_PAD, BN,
            [pre_x, x_l, *aggs], [w0a, w0b, *pnode[1:], w0rn, w0sn],
            [jnp.float32, jnp.float32, jnp.float32])

    dec = _prep_mlp(params["dec"])
    w2d = jnp.pad(dec[4], ((0, 0), (0, LAT - dec[4].shape[1])))
    b2d = jnp.pad(dec[5], ((0, 0), (0, LAT - dec[5].shape[1])))
    out = _call_rows(_dec_body, N_PAD, BN,
                     [pre_x], [dec[0], dec[1], dec[2], dec[3], w2d, b2d],
                     [jnp.float32])
    return out[:N, :3]
